# baseline (speedup 1.0000x reference)
import functools

import jax
import jax.numpy as jnp
from jax import lax
from jax.experimental import pallas as pl
from jax.experimental.pallas import tpu as pltpu
from jax.experimental.pallas import tpu_sc as plsc

IN_SIZE = 1000000
OUT_SIZE = 64
BATCH = 16384
HIST = 50
TOTAL = BATCH * HIST

_info = plsc.get_sparse_core_info()
_NC, _NS, _NL = _info.num_cores, _info.num_subcores, _info.num_lanes
NW = _NC * _NS
B_PER_W = TOTAL // NW
CHUNK = 256
NCHUNKS = B_PER_W // CHUNK
NPAIR = NCHUNKS // 2
NI = TOTAL // 128
NI_CHUNK = CHUNK // 128


@functools.partial(
    pl.kernel,
    mesh=plsc.VectorSubcoreMesh(core_axis_name="c", subcore_axis_name="s"),
    out_type=jax.ShapeDtypeStruct((8, NI, 8, 128), jnp.float32),
    scratch_types=[
        pltpu.VMEM((B_PER_W,), jnp.int32),
        pltpu.VMEM((CHUNK, OUT_SIZE), jnp.float32),
        pltpu.VMEM((CHUNK, OUT_SIZE), jnp.float32),
        pltpu.VMEM((8, NI_CHUNK, 8, 128), jnp.float32),
        pltpu.VMEM((8, NI_CHUNK, 8, 128), jnp.float32),
        pltpu.SemaphoreType.DMA,
        pltpu.SemaphoreType.DMA,
        pltpu.SemaphoreType.DMA,
        pltpu.SemaphoreType.DMA,
    ],
    compiler_params=pltpu.CompilerParams(
        use_tc_tiling_on_sc=False, needs_layout_passes=False
    ),
)
def _gather_kernel(
    table_hbm, idx_hbm, out_hbm, idx_v, rows_a, rows_b, t0, t1, ga, gb, wa, wb
):
    wid = lax.axis_index("s") * _NC + lax.axis_index("c")
    base = wid * B_PER_W
    rows = [rows_a, rows_b]
    gsem = [ga, gb]
    tbuf = [t0, t1]
    wsem = [wa, wb]

    pltpu.sync_copy(idx_hbm.at[pl.ds(base, B_PER_W)], idx_v)

    def g_copy(c, b):
        off = pl.multiple_of(c * CHUNK, 8)
        return pltpu.make_async_copy(
            table_hbm.at[idx_v.at[pl.ds(off, CHUNK)]], rows[b], gsem[b]
        )

    def w_copy(c, b):
        ig = pl.multiple_of((base + c * CHUNK) // 128, NI_CHUNK)
        return pltpu.make_async_copy(
            tbuf[b], out_hbm.at[:, pl.ds(ig, NI_CHUNK)], wsem[b]
        )

    lanes = jnp.arange(_NL, dtype=jnp.int32)

    def transpose(b):
        src = rows[b]
        dst = tbuf[b]

        @plsc.parallel_loop(0, CHUNK // _NL)
        def tbody(t):
            ridx = t * _NL + lanes
            iblk = t // 8
            i0 = (t % 8) * _NL
            for J in range(8):
                for j in range(8):
                    col = 8 * J + j
                    v = plsc.load_gather(
                        src, [ridx, jnp.full((_NL,), col, jnp.int32)]
                    )
                    dst[J, iblk, j, pl.ds(i0, _NL)] = v

    g_copy(0, 0).start()
    g_copy(1, 1).start()
    g_copy(0, 0).wait()
    transpose(0)
    w_copy(0, 0).start()
    g_copy(2, 0).start()
    g_copy(1, 1).wait()
    transpose(1)
    w_copy(1, 1).start()
    g_copy(3, 1).start()

    def body(i, carry):
        c = 2 * i
        g_copy(c, 0).wait()
        w_copy(c - 2, 0).wait()
        transpose(0)
        w_copy(c, 0).start()
        g_copy(c + 2, 0).start()
        g_copy(c + 1, 1).wait()
        w_copy(c - 1, 1).wait()
        transpose(1)
        w_copy(c + 1, 1).start()
        g_copy(c + 3, 1).start()
        return carry

    lax.fori_loop(1, NPAIR - 1, body, 0)

    c = NCHUNKS - 2
    g_copy(c, 0).wait()
    w_copy(c - 2, 0).wait()
    transpose(0)
    w_copy(c, 0).start()
    g_copy(c + 1, 1).wait()
    w_copy(c - 1, 1).wait()
    transpose(1)
    w_copy(c + 1, 1).start()
    w_copy(c, 0).wait()
    w_copy(c + 1, 1).wait()


def kernel(x, mat):
    flat_idx = x.reshape(-1)
    out4 = _gather_kernel(mat, flat_idx)
    return out4.transpose(1, 3, 0, 2).reshape(TOTAL, OUT_SIZE)


# speedup vs baseline: 1.7155x; 1.7155x over previous
import functools

import jax
import jax.numpy as jnp
from jax import lax
from jax.experimental import pallas as pl
from jax.experimental.pallas import tpu as pltpu
from jax.experimental.pallas import tpu_sc as plsc

IN_SIZE = 1000000
OUT_SIZE = 64
BATCH = 16384
HIST = 50
TOTAL = BATCH * HIST

_info = plsc.get_sparse_core_info()
_NC, _NS, _NL = _info.num_cores, _info.num_subcores, _info.num_lanes
NW = _NC * _NS
B_PER_W = TOTAL // NW
CHUNK = 256
NCHUNKS = B_PER_W // CHUNK
NPAIR = NCHUNKS // 2
NI = TOTAL // 128
NI_CHUNK = CHUNK // 128


@functools.partial(
    pl.kernel,
    mesh=plsc.VectorSubcoreMesh(core_axis_name="c", subcore_axis_name="s"),
    out_type=jax.ShapeDtypeStruct((8, NI, 8, 128), jnp.float32),
    scratch_types=[
        pltpu.VMEM((B_PER_W,), jnp.int32),
        pltpu.VMEM((CHUNK, OUT_SIZE), jnp.float32),
        pltpu.VMEM((CHUNK, OUT_SIZE), jnp.float32),
        pltpu.VMEM((8, NI_CHUNK, 8, 129), jnp.float32),
        pltpu.VMEM((8, NI_CHUNK, 8, 129), jnp.float32),
        pltpu.SemaphoreType.DMA,
        pltpu.SemaphoreType.DMA,
        pltpu.SemaphoreType.DMA,
        pltpu.SemaphoreType.DMA,
    ],
    compiler_params=pltpu.CompilerParams(
        use_tc_tiling_on_sc=False, needs_layout_passes=False
    ),
)
def _gather_kernel(
    table_hbm, idx_hbm, out_hbm, idx_v, rows_a, rows_b, t0, t1, ga, gb, wa, wb
):
    wid = lax.axis_index("s") * _NC + lax.axis_index("c")
    base = wid * B_PER_W
    rows = [rows_a, rows_b]
    gsem = [ga, gb]
    tbuf = [t0, t1]
    wsem = [wa, wb]

    pltpu.sync_copy(idx_hbm.at[pl.ds(base, B_PER_W)], idx_v)

    def g_copy(c, b):
        off = pl.multiple_of(c * CHUNK, 8)
        return pltpu.make_async_copy(
            table_hbm.at[idx_v.at[pl.ds(off, CHUNK)]], rows[b], gsem[b]
        )

    def w_copy(c, b):
        ig = pl.multiple_of((base + c * CHUNK) // 128, NI_CHUNK)
        return pltpu.make_async_copy(
            tbuf[b].at[:, :, :, pl.ds(0, 128)],
            out_hbm.at[:, pl.ds(ig, NI_CHUNK)],
            wsem[b],
        )

    lanes = jnp.arange(_NL, dtype=jnp.int32)
    lane_hi = lanes // 8
    lane_lo = lanes % 8

    def transpose(b):
        src = rows[b]
        dst = tbuf[b]

        @plsc.parallel_loop(0, CHUNK)
        def tbody(r):
            iblk_s = jnp.full((_NL,), r // 128, jnp.int32)
            i_s = jnp.full((_NL,), r % 128, jnp.int32)
            for k in range(OUT_SIZE // _NL):
                v = src[r, pl.ds(k * _NL, _NL)]
                plsc.store_scatter(
                    dst, [2 * k + lane_hi, iblk_s, lane_lo, i_s], v
                )

    g_copy(0, 0).start()
    g_copy(1, 1).start()
    g_copy(0, 0).wait()
    transpose(0)
    w_copy(0, 0).start()
    g_copy(2, 0).start()
    g_copy(1, 1).wait()
    transpose(1)
    w_copy(1, 1).start()
    g_copy(3, 1).start()

    def body(i, carry):
        c = 2 * i
        g_copy(c, 0).wait()
        w_copy(c - 2, 0).wait()
        transpose(0)
        w_copy(c, 0).start()
        g_copy(c + 2, 0).start()
        g_copy(c + 1, 1).wait()
        w_copy(c - 1, 1).wait()
        transpose(1)
        w_copy(c + 1, 1).start()
        g_copy(c + 3, 1).start()
        return carry

    lax.fori_loop(1, NPAIR - 1, body, 0)

    c = NCHUNKS - 2
    g_copy(c, 0).wait()
    w_copy(c - 2, 0).wait()
    transpose(0)
    w_copy(c, 0).start()
    g_copy(c + 1, 1).wait()
    w_copy(c - 1, 1).wait()
    transpose(1)
    w_copy(c + 1, 1).start()
    w_copy(c, 0).wait()
    w_copy(c + 1, 1).wait()


def kernel(x, mat):
    flat_idx = x.reshape(-1)
    out4 = _gather_kernel(mat, flat_idx)
    return out4.transpose(1, 3, 0, 2).reshape(TOTAL, OUT_SIZE)


# speedup vs baseline: 1.7158x; 1.0002x over previous
import functools

import jax
import jax.numpy as jnp
from jax import lax
from jax.experimental import pallas as pl
from jax.experimental.pallas import tpu as pltpu
from jax.experimental.pallas import tpu_sc as plsc

IN_SIZE = 1000000
OUT_SIZE = 64
BATCH = 16384
HIST = 50
TOTAL = BATCH * HIST

_info = plsc.get_sparse_core_info()
_NC, _NS, _NL = _info.num_cores, _info.num_subcores, _info.num_lanes
NW = _NC * _NS
B_PER_W = TOTAL // NW
CHUNK = 256
NCHUNKS = B_PER_W // CHUNK
NPAIR = NCHUNKS // 2
NI = TOTAL // 128
NI_CHUNK = CHUNK // 128


@functools.partial(
    pl.kernel,
    mesh=plsc.VectorSubcoreMesh(core_axis_name="c", subcore_axis_name="s"),
    out_type=jax.ShapeDtypeStruct((8, NI, 8, 128), jnp.float32),
    scratch_types=[
        pltpu.VMEM((B_PER_W,), jnp.int32),
        pltpu.VMEM((CHUNK, 2 * OUT_SIZE), jnp.float32),
        pltpu.VMEM((CHUNK, 2 * OUT_SIZE), jnp.float32),
        pltpu.VMEM((8, NI_CHUNK, 8, 129), jnp.float32),
        pltpu.VMEM((8, NI_CHUNK, 8, 129), jnp.float32),
        pltpu.SemaphoreType.DMA,
        pltpu.SemaphoreType.DMA,
        pltpu.SemaphoreType.DMA,
        pltpu.SemaphoreType.DMA,
    ],
    compiler_params=pltpu.CompilerParams(
        use_tc_tiling_on_sc=False, needs_layout_passes=False
    ),
)
def _gather_kernel(
    table_hbm, idx_hbm, out_hbm, idx_v, rows_a, rows_b, t0, t1, ga, gb, wa, wb
):
    wid = lax.axis_index("s") * _NC + lax.axis_index("c")
    base = wid * B_PER_W
    rows = [rows_a, rows_b]
    gsem = [ga, gb]
    tbuf = [t0, t1]
    wsem = [wa, wb]

    pltpu.sync_copy(idx_hbm.at[pl.ds(base, B_PER_W)], idx_v)

    def g_copy(c, b):
        off = pl.multiple_of(c * CHUNK, 8)
        return pltpu.make_async_copy(
            table_hbm.at[idx_v.at[pl.ds(off, CHUNK)]], rows[b], gsem[b]
        )

    def w_copy(c, b):
        ig = pl.multiple_of((base + c * CHUNK) // 128, NI_CHUNK)
        return pltpu.make_async_copy(
            tbuf[b].at[:, :, :, pl.ds(0, 128)],
            out_hbm.at[:, pl.ds(ig, NI_CHUNK)],
            wsem[b],
        )

    lanes = jnp.arange(_NL, dtype=jnp.int32)
    lane_hi = lanes // 8
    lane_lo = lanes % 8

    def transpose(b):
        src = rows[b]
        dst = tbuf[b]

        @plsc.parallel_loop(0, CHUNK)
        def tbody(r):
            iblk_s = jnp.full((_NL,), r // 128, jnp.int32)
            i_s = jnp.full((_NL,), r % 128, jnp.int32)
            for k in range(OUT_SIZE // _NL):
                v = src[r, pl.ds(k * _NL, _NL)]
                plsc.store_scatter(
                    dst, [2 * k + lane_hi, iblk_s, lane_lo, i_s], v
                )

    g_copy(0, 0).start()
    g_copy(1, 1).start()
    g_copy(0, 0).wait()
    transpose(0)
    w_copy(0, 0).start()
    g_copy(2, 0).start()
    g_copy(1, 1).wait()
    transpose(1)
    w_copy(1, 1).start()
    g_copy(3, 1).start()

    def body(i, carry):
        c = 2 * i
        g_copy(c, 0).wait()
        w_copy(c - 2, 0).wait()
        transpose(0)
        w_copy(c, 0).start()
        g_copy(c + 2, 0).start()
        g_copy(c + 1, 1).wait()
        w_copy(c - 1, 1).wait()
        transpose(1)
        w_copy(c + 1, 1).start()
        g_copy(c + 3, 1).start()
        return carry

    lax.fori_loop(1, NPAIR - 1, body, 0)

    c = NCHUNKS - 2
    g_copy(c, 0).wait()
    w_copy(c - 2, 0).wait()
    transpose(0)
    w_copy(c, 0).start()
    g_copy(c + 1, 1).wait()
    w_copy(c - 1, 1).wait()
    transpose(1)
    w_copy(c + 1, 1).start()
    w_copy(c, 0).wait()
    w_copy(c + 1, 1).wait()


def kernel(x, mat):
    flat_idx = x.reshape(-1)
    mat_p = jnp.pad(mat, ((0, 0), (0, OUT_SIZE)))
    out4 = _gather_kernel(mat_p, flat_idx)
    return out4.transpose(1, 3, 0, 2).reshape(TOTAL, OUT_SIZE)


# speedup vs baseline: 1.8570x; 1.0823x over previous
import functools

import jax
import jax.numpy as jnp
from jax import lax
from jax.experimental import pallas as pl
from jax.experimental.pallas import tpu as pltpu
from jax.experimental.pallas import tpu_sc as plsc

IN_SIZE = 1000000
OUT_SIZE = 64
BATCH = 16384
HIST = 50
TOTAL = BATCH * HIST

_info = plsc.get_sparse_core_info()
_NC, _NS, _NL = _info.num_cores, _info.num_subcores, _info.num_lanes
NW = _NC * _NS
B_PER_W = TOTAL // NW
CHUNK = 256
NCHUNKS = B_PER_W // CHUNK
NPAIR = NCHUNKS // 2
NI = TOTAL // 128
NI_CHUNK = CHUNK // 128


@functools.partial(
    pl.kernel,
    mesh=plsc.VectorSubcoreMesh(core_axis_name="c", subcore_axis_name="s"),
    out_type=jax.ShapeDtypeStruct((8, NI, 8, 128), jnp.float32),
    scratch_types=[
        pltpu.VMEM((B_PER_W,), jnp.int32),
        pltpu.VMEM((CHUNK, OUT_SIZE), jnp.float32),
        pltpu.VMEM((CHUNK, OUT_SIZE), jnp.float32),
        pltpu.VMEM((8, NI_CHUNK, 8, 129), jnp.float32),
        pltpu.VMEM((8, NI_CHUNK, 8, 129), jnp.float32),
        pltpu.SemaphoreType.DMA,
        pltpu.SemaphoreType.DMA,
        pltpu.SemaphoreType.DMA,
        pltpu.SemaphoreType.DMA,
    ],
    compiler_params=pltpu.CompilerParams(
        use_tc_tiling_on_sc=False, needs_layout_passes=False
    ),
)
def _gather_kernel(
    table_hbm, idx_hbm, out_hbm, idx_v, rows_a, rows_b, t0, t1, ga, gb, wa, wb
):
    wid = lax.axis_index("s") * _NC + lax.axis_index("c")
    base = wid * B_PER_W
    rows = [rows_a, rows_b]
    gsem = [ga, gb]
    tbuf = [t0, t1]
    wsem = [wa, wb]

    pltpu.sync_copy(idx_hbm.at[pl.ds(base, B_PER_W)], idx_v)

    @plsc.parallel_loop(0, B_PER_W // _NL)
    def _dbl(i):
        sl = pl.ds(i * _NL, _NL)
        idx_v[sl] = idx_v[sl] * 2

    def g_copy(c, b):
        off = pl.multiple_of(c * CHUNK, 8)
        return pltpu.make_async_copy(
            table_hbm.at[idx_v.at[pl.ds(off, CHUNK)]], rows[b], gsem[b]
        )

    def w_copy(c, b):
        ig = pl.multiple_of((base + c * CHUNK) // 128, NI_CHUNK)
        return pltpu.make_async_copy(
            tbuf[b].at[:, :, :, pl.ds(0, 128)],
            out_hbm.at[:, pl.ds(ig, NI_CHUNK)],
            wsem[b],
        )

    lanes = jnp.arange(_NL, dtype=jnp.int32)
    lane_hi = lanes // 8
    lane_lo = lanes % 8

    def transpose(b):
        src = rows[b]
        dst = tbuf[b]

        @plsc.parallel_loop(0, CHUNK)
        def tbody(r):
            iblk_s = jnp.full((_NL,), r // 128, jnp.int32)
            i_s = jnp.full((_NL,), r % 128, jnp.int32)
            for k in range(OUT_SIZE // _NL):
                v = src[r, pl.ds(k * _NL, _NL)]
                plsc.store_scatter(
                    dst, [2 * k + lane_hi, iblk_s, lane_lo, i_s], v
                )

    g_copy(0, 0).start()
    g_copy(1, 1).start()
    g_copy(0, 0).wait()
    transpose(0)
    w_copy(0, 0).start()
    g_copy(2, 0).start()
    g_copy(1, 1).wait()
    transpose(1)
    w_copy(1, 1).start()
    g_copy(3, 1).start()

    def body(i, carry):
        c = 2 * i
        g_copy(c, 0).wait()
        w_copy(c - 2, 0).wait()
        transpose(0)
        w_copy(c, 0).start()
        g_copy(c + 2, 0).start()
        g_copy(c + 1, 1).wait()
        w_copy(c - 1, 1).wait()
        transpose(1)
        w_copy(c + 1, 1).start()
        g_copy(c + 3, 1).start()
        return carry

    lax.fori_loop(1, NPAIR - 1, body, 0)

    c = NCHUNKS - 2
    g_copy(c, 0).wait()
    w_copy(c - 2, 0).wait()
    transpose(0)
    w_copy(c, 0).start()
    g_copy(c + 1, 1).wait()
    w_copy(c - 1, 1).wait()
    transpose(1)
    w_copy(c + 1, 1).start()
    w_copy(c, 0).wait()
    w_copy(c + 1, 1).wait()


def kernel(x, mat):
    flat_idx = x.reshape(-1)
    mat_p = jnp.pad(mat, ((0, 0), (0, OUT_SIZE)))
    mat2 = mat_p.reshape(2 * IN_SIZE, OUT_SIZE)
    out4 = _gather_kernel(mat2, flat_idx)
    return out4.transpose(1, 3, 0, 2).reshape(TOTAL, OUT_SIZE)
